# baseline (speedup 1.0000x reference)
import functools

import jax
import jax.numpy as jnp
from jax.experimental import pallas as pl
from jax.experimental.pallas import tpu as pltpu

HIDDEN = 1024
NUM_EXPERTS = 8
TOP_K = 2
N_TOKENS = 4096

BMG = 256
P_MAX = 2 * N_TOKENS + NUM_EXPERTS * BMG
NB = P_MAX // BMG


def _topk_masked_gates(logits):
    m = jnp.max(logits, axis=-1, keepdims=True)
    ex = jnp.exp(logits - m)
    g = ex / jnp.sum(ex, axis=-1, keepdims=True)
    ids = jax.lax.broadcasted_iota(jnp.int32, g.shape, 1)
    m1 = jnp.max(g, axis=-1, keepdims=True)
    a1 = jnp.min(jnp.where(g == m1, ids, NUM_EXPERTS), axis=-1, keepdims=True)
    g_wo1 = jnp.where(ids == a1, -jnp.inf, g)
    m2 = jnp.max(g_wo1, axis=-1, keepdims=True)
    a2 = jnp.min(jnp.where(g_wo1 == m2, ids, NUM_EXPERTS), axis=-1,
                 keepdims=True)
    keep = (ids == a1) | (ids == a2)
    return jnp.where(keep, g, 0.0)


def _gate_body(x_ref, wg_ref, bg_ref, gmt_ref):
    logits = jnp.dot(x_ref[...], wg_ref[...],
                     preferred_element_type=jnp.float32) + bg_ref[...]
    gm = _topk_masked_gates(logits)
    gmt_ref[...] = gm.T


def _stage_a(x, W_gate, b_gate):
    bm = 1024
    return pl.pallas_call(
        _gate_body,
        grid=(N_TOKENS // bm,),
        in_specs=[
            pl.BlockSpec((bm, HIDDEN), lambda t: (t, 0)),
            pl.BlockSpec((HIDDEN, NUM_EXPERTS), lambda t: (0, 0)),
            pl.BlockSpec((1, NUM_EXPERTS), lambda t: (0, 0)),
        ],
        out_specs=pl.BlockSpec((NUM_EXPERTS, bm), lambda t: (0, t)),
        out_shape=jax.ShapeDtypeStruct((NUM_EXPERTS, N_TOKENS), jnp.float32),
    )(x, W_gate, b_gate.reshape(1, NUM_EXPERTS))


def _gmm_body(be_ref, xs_ref, w_ref, b_ref, y_ref):
    y_ref[...] = (jnp.dot(xs_ref[...], w_ref[0],
                          preferred_element_type=jnp.float32) + b_ref[0])


def _stage_c(x_sorted, W_experts, b_experts, block_expert):
    grid_spec = pltpu.PrefetchScalarGridSpec(
        num_scalar_prefetch=1,
        grid=(NB,),
        in_specs=[
            pl.BlockSpec((BMG, HIDDEN), lambda i, be: (i, 0)),
            pl.BlockSpec((1, HIDDEN, HIDDEN), lambda i, be: (be[i], 0, 0)),
            pl.BlockSpec((1, 1, HIDDEN), lambda i, be: (be[i], 0, 0)),
        ],
        out_specs=pl.BlockSpec((BMG, HIDDEN), lambda i, be: (i, 0)),
    )
    return pl.pallas_call(
        _gmm_body,
        grid_spec=grid_spec,
        out_shape=jax.ShapeDtypeStruct((P_MAX, HIDDEN), jnp.float32),
        compiler_params=pltpu.CompilerParams(
            dimension_semantics=("arbitrary",),
        ),
    )(block_expert, x_sorted, W_experts,
      b_experts.reshape(NUM_EXPERTS, 1, HIDDEN))


def _routing_jnp(g_mT):
    gm = g_mT.T
    ids = jnp.arange(NUM_EXPERTS, dtype=jnp.int32)[None, :]
    m1 = jnp.max(gm, axis=-1)
    a1 = jnp.argmax(gm, axis=-1).astype(jnp.int32)
    gm2 = jnp.where(ids == a1[:, None], -jnp.inf, gm)
    m2 = jnp.max(gm2, axis=-1)
    a2 = jnp.argmax(gm2, axis=-1).astype(jnp.int32)

    e_all = jnp.concatenate([a1, a2])
    counts = jnp.sum(
        (e_all[:, None] == ids).astype(jnp.int32), axis=0)
    padded = ((counts + BMG - 1) // BMG) * BMG
    off_pad = jnp.concatenate(
        [jnp.zeros((1,), jnp.int32), jnp.cumsum(padded)[:-1]])
    off_raw = jnp.concatenate(
        [jnp.zeros((1,), jnp.int32), jnp.cumsum(counts)[:-1]])
    order = jnp.argsort(e_all, stable=True)
    e_sorted = e_all[order]
    dest_of_sorted = off_pad[e_sorted] + (
        jnp.arange(2 * N_TOKENS, dtype=jnp.int32) - off_raw[e_sorted])
    dest_pos = jnp.zeros((2 * N_TOKENS,), jnp.int32).at[order].set(
        dest_of_sorted)
    blk_ids = jnp.arange(NB, dtype=jnp.int32) * BMG
    block_expert = jnp.sum(
        (blk_ids[:, None] >= off_pad[None, :]).astype(jnp.int32), axis=1) - 1
    return dest_pos, block_expert, m1, m2


@jax.jit
def kernel(x, W_experts, b_experts, W_gate, b_gate):
    g_mT = _stage_a(x, W_gate, b_gate)
    dest_pos, block_expert, w1, w2 = _routing_jnp(g_mT)
    tok = jnp.concatenate([jnp.arange(N_TOKENS, dtype=jnp.int32)] * 2)
    x_sorted = jnp.zeros((P_MAX, HIDDEN), jnp.float32).at[dest_pos].set(
        x[tok])
    y = _stage_c(x_sorted, W_experts, b_experts, block_expert)
    out = (w1[:, None] * y[dest_pos[:N_TOKENS]]
           + w2[:, None] * y[dest_pos[N_TOKENS:]])
    return out


# speedup vs baseline: 1.5755x; 1.5755x over previous
import functools

import jax
import jax.numpy as jnp
from jax import lax
from jax.experimental import pallas as pl
from jax.experimental.pallas import tpu as pltpu
from jax.experimental.pallas import tpu_sc as plsc

HIDDEN = 1024
NUM_EXPERTS = 8
TOP_K = 2
N_TOKENS = 4096

BMG = 256
P_MAX = 2 * N_TOKENS + NUM_EXPERTS * BMG
NB = P_MAX // BMG
NB_PAD = 64

NC = 2
NS = 16
L = 16
CH = (2 * N_TOKENS) // NS
NG = CH // L


def _topk_masked_gates(logits):
    m = jnp.max(logits, axis=-1, keepdims=True)
    ex = jnp.exp(logits - m)
    g = ex / jnp.sum(ex, axis=-1, keepdims=True)
    ids = jax.lax.broadcasted_iota(jnp.int32, g.shape, 1)
    m1 = jnp.max(g, axis=-1, keepdims=True)
    a1 = jnp.min(jnp.where(g == m1, ids, NUM_EXPERTS), axis=-1, keepdims=True)
    g_wo1 = jnp.where(ids == a1, -jnp.inf, g)
    m2 = jnp.max(g_wo1, axis=-1, keepdims=True)
    a2 = jnp.min(jnp.where(g_wo1 == m2, ids, NUM_EXPERTS), axis=-1,
                 keepdims=True)
    keep = (ids == a1) | (ids == a2)
    return jnp.where(keep, g, 0.0)


def _gate_body(x_ref, wg_ref, bg_ref, gmt_ref):
    logits = jnp.dot(x_ref[...], wg_ref[...],
                     preferred_element_type=jnp.float32) + bg_ref[...]
    gm = _topk_masked_gates(logits)
    gmt_ref[...] = gm.T


def _stage_a(x, W_gate, b_gate):
    bm = 1024
    return pl.pallas_call(
        _gate_body,
        grid=(N_TOKENS // bm,),
        in_specs=[
            pl.BlockSpec((bm, HIDDEN), lambda t: (t, 0)),
            pl.BlockSpec((HIDDEN, NUM_EXPERTS), lambda t: (0, 0)),
            pl.BlockSpec((1, NUM_EXPERTS), lambda t: (0, 0)),
        ],
        out_specs=pl.BlockSpec((NUM_EXPERTS, bm), lambda t: (0, t)),
        out_shape=jax.ShapeDtypeStruct((NUM_EXPERTS, N_TOKENS), jnp.float32),
    )(x, W_gate, b_gate.reshape(1, NUM_EXPERTS))


def _lane_iota():
    return jax.lax.broadcasted_iota(jnp.int32, (L,), 0)


def _bc(x, dtype=jnp.int32):
    return jnp.broadcast_to(jnp.asarray(x, dtype), (L,))


def _slot_expert(gbuf, i, slot):
    gs = [gbuf[e, pl.ds(i * L, L)] for e in range(NUM_EXPERTS)]
    m1 = gs[0]
    a1 = jnp.zeros((L,), jnp.int32)
    for e in range(1, NUM_EXPERTS):
        gt = gs[e] > m1
        m1 = jnp.where(gt, gs[e], m1)
        a1 = jnp.where(gt, _bc(e), a1)
    neg = jnp.full((L,), -jnp.inf, jnp.float32)
    m2 = jnp.where(a1 == jnp.zeros((L,), jnp.int32), neg, gs[0])
    a2 = jnp.zeros((L,), jnp.int32)
    for e in range(1, NUM_EXPERTS):
        ge = jnp.where(a1 == _bc(e), neg, gs[e])
        gt = ge > m2
        m2 = jnp.where(gt, ge, m2)
        a2 = jnp.where(gt, _bc(e), a2)
    ew = jnp.where(_bc(slot) == jnp.zeros((L,), jnp.int32), a1, a2)
    return ew, m1, m2


def _stage_b_kernel(gmt_hbm, x_hbm, dest_hbm, be_hbm, xs_hbm, hist_hbm,
                    gbuf, ebuf, dbuf, vbuf, tabbuf, bebuf,
                    rbuf0, rbuf1,
                    semg0, semg1, sems0, sems1, semc):
    c = lax.axis_index("c")
    s = lax.axis_index("s")
    slot = s // 8
    t0 = (s % 8) * CH
    lanes = _lane_iota()

    for e in range(NUM_EXPERTS):
        pltpu.sync_copy(gmt_hbm.at[e, pl.ds(t0, CH)], gbuf.at[e])
    if _BCUT == 1:
        return

    def p1_body(i, hist):
        ew, _, _ = _slot_expert(gbuf, i, slot)
        ebuf[pl.ds(i * L, L)] = ew
        for e in range(NUM_EXPERTS):
            ne = jnp.sum((ew == _bc(e)).astype(jnp.int32), axis=0)
            hist = hist + jnp.where(lanes == _bc(e), _bc(ne),
                                    jnp.zeros((L,), jnp.int32))
        return hist

    hist = lax.fori_loop(0, NG, p1_body, jnp.zeros((L,), jnp.int32),
                         unroll=False)
    vbuf[...] = hist
    if _BCUT == 2:
        return
    pltpu.sync_copy(vbuf, hist_hbm.at[s])
    plsc.subcore_barrier()
    pltpu.sync_copy(hist_hbm, tabbuf)
    if _BCUT == 3:
        return

    rows = [tabbuf[w] for w in range(NS)]
    tot = rows[0]
    for w in range(1, NS):
        tot = tot + rows[w]
    pad = jnp.bitwise_and(tot + (BMG - 1), -BMG)
    off = plsc.cumsum(pad) - pad
    base = off
    for w in range(NS):
        mk = _bc((w < s).astype(jnp.int32))
        base = base + rows[w] * mk

    zero_v = jnp.zeros((L,), jnp.int32)
    off_sc = [jnp.sum(jnp.where(lanes == _bc(e), off, zero_v), axis=0)
              for e in range(NUM_EXPERTS)]
    for k in range(NB_PAD // L):
        iv = (lanes + k * L) * BMG
        bev = jnp.full((L,), -1, jnp.int32)
        for e in range(NUM_EXPERTS):
            bev = bev + (iv >= _bc(off_sc[e])).astype(jnp.int32)
        bebuf[pl.ds(k * L, L)] = bev

    @pl.when(jnp.logical_and(c == 0, s == 0))
    def _():
        pltpu.sync_copy(bebuf, be_hbm)

    if _BCUT == 4:
        return

    def p2_body(i, run):
        ew = ebuf[pl.ds(i * L, L)]
        dest = jnp.zeros((L,), jnp.int32)
        zv = jnp.zeros((L,), jnp.int32)
        for e in range(NUM_EXPERTS):
            mk = ew == _bc(e)
            mi = mk.astype(jnp.int32)
            pref = plsc.cumsum(mi) - mi
            run_e = jnp.sum(jnp.where(lanes == _bc(e), run, zv), axis=0)
            dest = jnp.where(mk, _bc(run_e) + pref, dest)
            run = run + jnp.where(lanes == _bc(e), _bc(jnp.sum(mi, axis=0)),
                                  zv)
        dbuf[pl.ds(i * L, L)] = dest
        return run

    lax.fori_loop(0, NG, p2_body, base, unroll=False)

    @pl.when(c == 0)
    def _():
        pltpu.sync_copy(dbuf, dest_hbm.at[pl.ds(s * CH, CH)])

    if _NO_PHASE3:
        return
    half = CH // 2
    tbase = t0 + c * half
    dofs = c * half
    rbufs = (rbuf0, rbuf1)
    semgs = (semg0, semg1)
    semss = (sems0, sems1)
    hscat = [None, None]
    for i in range(half // L):
        b = i % 2
        if hscat[b] is not None:
            hscat[b].wait()
        tok16 = _bc(tbase) + i * L + lanes
        pltpu.async_copy(x_hbm.at[tok16], rbufs[b], semgs[b]).wait()
        dest16 = dbuf[pl.ds(dofs + i * L, L)]
        hscat[b] = pltpu.async_copy(rbufs[b], xs_hbm.at[dest16], semss[b])
    for h in hscat:
        if h is not None:
            h.wait()


def _stage_b(g_mT, x):
    mesh = plsc.VectorSubcoreMesh(core_axis_name="c", subcore_axis_name="s")
    return pl.kernel(
        _stage_b_kernel,
        mesh=mesh,
        compiler_params=pltpu.CompilerParams(needs_layout_passes=False),
        out_type=[
            jax.ShapeDtypeStruct((2 * N_TOKENS,), jnp.int32),
            jax.ShapeDtypeStruct((NB_PAD,), jnp.int32),
            jax.ShapeDtypeStruct((P_MAX, HIDDEN), jnp.float32),
            jax.ShapeDtypeStruct((NS, L), jnp.int32),
        ],
        scratch_types=[
            pltpu.VMEM((NUM_EXPERTS, CH), jnp.float32),
            pltpu.VMEM((CH,), jnp.int32),
            pltpu.VMEM((CH,), jnp.int32),
            pltpu.VMEM((L,), jnp.int32),
            pltpu.VMEM((NS, L), jnp.int32),
            pltpu.VMEM((NB_PAD,), jnp.int32),
            pltpu.VMEM((L, HIDDEN), jnp.float32),
            pltpu.VMEM((L, HIDDEN), jnp.float32),
            pltpu.SemaphoreType.DMA,
            pltpu.SemaphoreType.DMA,
            pltpu.SemaphoreType.DMA,
            pltpu.SemaphoreType.DMA,
            pltpu.SemaphoreType.DMA,
        ],
    )(g_mT, x)


def _gmm_body(be_ref, xs_ref, w_ref, b_ref, y_ref):
    y_ref[...] = (jnp.dot(xs_ref[...], w_ref[0],
                          preferred_element_type=jnp.float32) + b_ref[0])


def _stage_c(x_sorted, W_experts, b_experts, block_expert):
    grid_spec = pltpu.PrefetchScalarGridSpec(
        num_scalar_prefetch=1,
        grid=(NB,),
        in_specs=[
            pl.BlockSpec((BMG, HIDDEN), lambda i, be: (i, 0)),
            pl.BlockSpec((1, HIDDEN, HIDDEN), lambda i, be: (be[i], 0, 0)),
            pl.BlockSpec((1, 1, HIDDEN), lambda i, be: (be[i], 0, 0)),
        ],
        out_specs=pl.BlockSpec((BMG, HIDDEN), lambda i, be: (i, 0)),
    )
    return pl.pallas_call(
        _gmm_body,
        grid_spec=grid_spec,
        out_shape=jax.ShapeDtypeStruct((P_MAX, HIDDEN), jnp.float32),
        compiler_params=pltpu.CompilerParams(
            dimension_semantics=("arbitrary",),
        ),
    )(block_expert, x_sorted, W_experts,
      b_experts.reshape(NUM_EXPERTS, 1, HIDDEN))


TPW = N_TOKENS // (NC * NS)
VPR = HIDDEN // L


def _stage_d_kernel(gmt_hbm, dest_hbm, y_hbm, out_hbm,
                    gbuf, d0buf, d1buf, w1buf, w2buf,
                    y0buf, y1buf, obuf, semg0, semg1):
    c = lax.axis_index("c")
    s = lax.axis_index("s")
    wid = s * NC + c
    t0 = wid * TPW
    lanes = _lane_iota()

    for e in range(NUM_EXPERTS):
        pltpu.sync_copy(gmt_hbm.at[e, pl.ds(t0, TPW)], gbuf.at[e])
    pltpu.sync_copy(dest_hbm.at[pl.ds(t0, TPW)], d0buf)
    pltpu.sync_copy(dest_hbm.at[pl.ds(N_TOKENS + t0, TPW)], d1buf)

    def wts_body(i, carry):
        _, m1, m2 = _slot_expert(gbuf, i, 0)
        w1buf[pl.ds(i * L, L)] = m1
        w2buf[pl.ds(i * L, L)] = m2
        return carry

    lax.fori_loop(0, TPW // L, wts_body, 0, unroll=False)

    for k in range(TPW // L):
        d016 = d0buf[pl.ds(k * L, L)]
        d116 = d1buf[pl.ds(k * L, L)]
        pltpu.async_copy(y_hbm.at[d016], y0buf, semg0).wait()
        pltpu.async_copy(y_hbm.at[d116], y1buf, semg1).wait()
        w1v = w1buf[pl.ds(k * L, L)]
        w2v = w2buf[pl.ds(k * L, L)]
        for tl in range(L):
            w1 = _bc(w1v[tl], jnp.float32)
            w2 = _bc(w2v[tl], jnp.float32)

            def row_body(v, carry2, tl=tl, w1=w1, w2=w2):
                obuf[tl, pl.ds(v * L, L)] = (
                    w1 * y0buf[tl, pl.ds(v * L, L)]
                    + w2 * y1buf[tl, pl.ds(v * L, L)])
                return carry2

            lax.fori_loop(0, VPR, row_body, 0, unroll=False)
        pltpu.sync_copy(obuf, out_hbm.at[pl.ds(t0 + k * L, L)])


def _stage_d(g_mT, dest_pos, y):
    mesh = plsc.VectorSubcoreMesh(core_axis_name="c", subcore_axis_name="s")
    return pl.kernel(
        _stage_d_kernel,
        mesh=mesh,
        compiler_params=pltpu.CompilerParams(needs_layout_passes=False),
        out_type=[jax.ShapeDtypeStruct((N_TOKENS, HIDDEN), jnp.float32)],
        scratch_types=[
            pltpu.VMEM((NUM_EXPERTS, TPW), jnp.float32),
            pltpu.VMEM((TPW,), jnp.int32),
            pltpu.VMEM((TPW,), jnp.int32),
            pltpu.VMEM((TPW,), jnp.float32),
            pltpu.VMEM((TPW,), jnp.float32),
            pltpu.VMEM((L, HIDDEN), jnp.float32),
            pltpu.VMEM((L, HIDDEN), jnp.float32),
            pltpu.VMEM((L, HIDDEN), jnp.float32),
            pltpu.SemaphoreType.DMA,
            pltpu.SemaphoreType.DMA,
        ],
    )(g_mT, dest_pos, y)


_DEBUG_STAGE = ""
_NO_PHASE3 = False
_BCUT = 99


_JNP_SHUFFLE = False
_JNP_COMBINE = False


@jax.jit
def kernel(x, W_experts, b_experts, W_gate, b_gate):
    g_mT = _stage_a(x, W_gate, b_gate)
    dest_pos, block_expert, x_sorted, _hist_tab = _stage_b(g_mT, x)
    if _JNP_SHUFFLE:
        tok = jnp.concatenate([jnp.arange(N_TOKENS, dtype=jnp.int32)] * 2)
        x_sorted = jnp.zeros((P_MAX, HIDDEN), jnp.float32).at[dest_pos].set(
            x[tok])
    y = _stage_c(x_sorted, W_experts, b_experts, block_expert)
    if _JNP_COMBINE:
        gm = g_mT.T
        ids = jnp.arange(NUM_EXPERTS, dtype=jnp.int32)[None, :]
        m1 = jnp.max(gm, axis=-1)
        a1 = jnp.argmax(gm, axis=-1).astype(jnp.int32)
        gm2 = jnp.where(ids == a1[:, None], -jnp.inf, gm)
        m2 = jnp.max(gm2, axis=-1)
        return (m1[:, None] * y[dest_pos[:N_TOKENS]]
                + m2[:, None] * y[dest_pos[N_TOKENS:]])
    out = _stage_d(g_mT, dest_pos, y)
    return out[0] if isinstance(out, (list, tuple)) else out


# speedup vs baseline: 1.8049x; 1.1456x over previous
import functools

import jax
import jax.numpy as jnp
from jax import lax
from jax.experimental import pallas as pl
from jax.experimental.pallas import tpu as pltpu
from jax.experimental.pallas import tpu_sc as plsc

HIDDEN = 1024
NUM_EXPERTS = 8
TOP_K = 2
N_TOKENS = 4096

BMG = 256
P_MAX = 2 * N_TOKENS + NUM_EXPERTS * BMG
NB = P_MAX // BMG
NB_PAD = 64

NC = 2
NS = 16
L = 16
CH = (2 * N_TOKENS) // NS
NG = CH // L


def _topk_masked_gates(logits):
    m = jnp.max(logits, axis=-1, keepdims=True)
    ex = jnp.exp(logits - m)
    g = ex / jnp.sum(ex, axis=-1, keepdims=True)
    ids = jax.lax.broadcasted_iota(jnp.int32, g.shape, 1)
    m1 = jnp.max(g, axis=-1, keepdims=True)
    a1 = jnp.min(jnp.where(g == m1, ids, NUM_EXPERTS), axis=-1, keepdims=True)
    g_wo1 = jnp.where(ids == a1, -jnp.inf, g)
    m2 = jnp.max(g_wo1, axis=-1, keepdims=True)
    a2 = jnp.min(jnp.where(g_wo1 == m2, ids, NUM_EXPERTS), axis=-1,
                 keepdims=True)
    keep = (ids == a1) | (ids == a2)
    return jnp.where(keep, g, 0.0)


def _gate_body(x_ref, wg_ref, bg_ref, gmt_ref):
    logits = jnp.dot(x_ref[...], wg_ref[...],
                     preferred_element_type=jnp.float32) + bg_ref[...]
    gm = _topk_masked_gates(logits)
    gmt_ref[...] = gm.T


def _stage_a(x, W_gate, b_gate):
    bm = 1024
    return pl.pallas_call(
        _gate_body,
        grid=(N_TOKENS // bm,),
        in_specs=[
            pl.BlockSpec((bm, HIDDEN), lambda t: (t, 0)),
            pl.BlockSpec((HIDDEN, NUM_EXPERTS), lambda t: (0, 0)),
            pl.BlockSpec((1, NUM_EXPERTS), lambda t: (0, 0)),
        ],
        out_specs=pl.BlockSpec((NUM_EXPERTS, bm), lambda t: (0, t)),
        out_shape=jax.ShapeDtypeStruct((NUM_EXPERTS, N_TOKENS), jnp.float32),
    )(x, W_gate, b_gate.reshape(1, NUM_EXPERTS))


def _lane_iota():
    return jax.lax.broadcasted_iota(jnp.int32, (L,), 0)


def _bc(x, dtype=jnp.int32):
    return jnp.broadcast_to(jnp.asarray(x, dtype), (L,))


def _slot_expert(gbuf, i, slot):
    gs = [gbuf[e, pl.ds(i * L, L)] for e in range(NUM_EXPERTS)]
    m1 = gs[0]
    a1 = jnp.zeros((L,), jnp.int32)
    for e in range(1, NUM_EXPERTS):
        gt = gs[e] > m1
        m1 = jnp.where(gt, gs[e], m1)
        a1 = jnp.where(gt, _bc(e), a1)
    neg = jnp.full((L,), -jnp.inf, jnp.float32)
    m2 = jnp.where(a1 == jnp.zeros((L,), jnp.int32), neg, gs[0])
    a2 = jnp.zeros((L,), jnp.int32)
    for e in range(1, NUM_EXPERTS):
        ge = jnp.where(a1 == _bc(e), neg, gs[e])
        gt = ge > m2
        m2 = jnp.where(gt, ge, m2)
        a2 = jnp.where(gt, _bc(e), a2)
    ew = jnp.where(_bc(slot) == jnp.zeros((L,), jnp.int32), a1, a2)
    return ew, m1, m2


def _stage_b_kernel(gmt_hbm, x_hbm, dest_hbm, be_hbm, xs_hbm, hist_hbm,
                    gbuf, ebuf, dbuf, vbuf, tabbuf, bebuf,
                    rbuf0, rbuf1,
                    semg0, semg1, sems0, sems1, semc):
    c = lax.axis_index("c")
    s = lax.axis_index("s")
    slot = s // 8
    t0 = (s % 8) * CH
    lanes = _lane_iota()

    for e in range(NUM_EXPERTS):
        pltpu.sync_copy(gmt_hbm.at[e, pl.ds(t0, CH)], gbuf.at[e])
    if _BCUT == 1:
        return

    def p1_body(i, hist):
        ew, _, _ = _slot_expert(gbuf, i, slot)
        ebuf[pl.ds(i * L, L)] = ew
        for e in range(NUM_EXPERTS):
            ne = jnp.sum((ew == _bc(e)).astype(jnp.int32), axis=0)
            hist = hist + jnp.where(lanes == _bc(e), _bc(ne),
                                    jnp.zeros((L,), jnp.int32))
        return hist

    hist = lax.fori_loop(0, NG, p1_body, jnp.zeros((L,), jnp.int32),
                         unroll=False)
    vbuf[...] = hist
    if _BCUT == 2:
        return
    pltpu.sync_copy(vbuf, hist_hbm.at[s])
    plsc.subcore_barrier()
    pltpu.sync_copy(hist_hbm, tabbuf)
    if _BCUT == 3:
        return

    rows = [tabbuf[w] for w in range(NS)]
    tot = rows[0]
    for w in range(1, NS):
        tot = tot + rows[w]
    pad = jnp.bitwise_and(tot + (BMG - 1), -BMG)
    off = plsc.cumsum(pad) - pad
    base = off
    for w in range(NS):
        mk = _bc((w < s).astype(jnp.int32))
        base = base + rows[w] * mk

    zero_v = jnp.zeros((L,), jnp.int32)
    off_sc = [jnp.sum(jnp.where(lanes == _bc(e), off, zero_v), axis=0)
              for e in range(NUM_EXPERTS)]
    for k in range(NB_PAD // L):
        iv = (lanes + k * L) * BMG
        bev = jnp.full((L,), -1, jnp.int32)
        for e in range(NUM_EXPERTS):
            bev = bev + (iv >= _bc(off_sc[e])).astype(jnp.int32)
        bebuf[pl.ds(k * L, L)] = bev

    @pl.when(jnp.logical_and(c == 0, s == 0))
    def _():
        pltpu.sync_copy(bebuf, be_hbm)

    if _BCUT == 4:
        return

    def p2_body(i, run):
        ew = ebuf[pl.ds(i * L, L)]
        dest = jnp.zeros((L,), jnp.int32)
        zv = jnp.zeros((L,), jnp.int32)
        for e in range(NUM_EXPERTS):
            mk = ew == _bc(e)
            mi = mk.astype(jnp.int32)
            pref = plsc.cumsum(mi) - mi
            run_e = jnp.sum(jnp.where(lanes == _bc(e), run, zv), axis=0)
            dest = jnp.where(mk, _bc(run_e) + pref, dest)
            run = run + jnp.where(lanes == _bc(e), _bc(jnp.sum(mi, axis=0)),
                                  zv)
        dbuf[pl.ds(i * L, L)] = dest
        return run

    lax.fori_loop(0, NG, p2_body, base, unroll=False)

    @pl.when(c == 0)
    def _():
        pltpu.sync_copy(dbuf, dest_hbm.at[pl.ds(s * CH, CH)])

    half = CH // 2
    tbase = t0 + c * half
    dofs = c * half
    rbufs = (rbuf0, rbuf1)
    semgs = (semg0, semg1)
    semss = (sems0, sems1)
    nch = half // L
    hg = [None, None]
    hscat = [None, None]
    hg[0] = pltpu.async_copy(x_hbm.at[pl.ds(tbase, L)], rbufs[0], semgs[0])
    for i in range(nch):
        b = i % 2
        nb = (i + 1) % 2
        if i + 1 < nch:
            if hscat[nb] is not None:
                hscat[nb].wait()
            hg[nb] = pltpu.async_copy(
                x_hbm.at[pl.ds(tbase + (i + 1) * L, L)], rbufs[nb],
                semgs[nb])
        hg[b].wait()
        dest16 = dbuf[pl.ds(dofs + i * L, L)]
        hscat[b] = pltpu.async_copy(rbufs[b], xs_hbm.at[dest16], semss[b])
    for h in hscat:
        if h is not None:
            h.wait()


def _stage_b(g_mT, x):
    mesh = plsc.VectorSubcoreMesh(core_axis_name="c", subcore_axis_name="s")
    return pl.kernel(
        _stage_b_kernel,
        mesh=mesh,
        compiler_params=pltpu.CompilerParams(needs_layout_passes=False),
        out_type=[
            jax.ShapeDtypeStruct((2 * N_TOKENS,), jnp.int32),
            jax.ShapeDtypeStruct((NB_PAD,), jnp.int32),
            jax.ShapeDtypeStruct((P_MAX, HIDDEN), jnp.float32),
            jax.ShapeDtypeStruct((NS, L), jnp.int32),
        ],
        scratch_types=[
            pltpu.VMEM((NUM_EXPERTS, CH), jnp.float32),
            pltpu.VMEM((CH,), jnp.int32),
            pltpu.VMEM((CH,), jnp.int32),
            pltpu.VMEM((L,), jnp.int32),
            pltpu.VMEM((NS, L), jnp.int32),
            pltpu.VMEM((NB_PAD,), jnp.int32),
            pltpu.VMEM((L, HIDDEN), jnp.float32),
            pltpu.VMEM((L, HIDDEN), jnp.float32),
            pltpu.SemaphoreType.DMA,
            pltpu.SemaphoreType.DMA,
            pltpu.SemaphoreType.DMA,
            pltpu.SemaphoreType.DMA,
            pltpu.SemaphoreType.DMA,
        ],
    )(g_mT, x)


def _gmm_body(be_ref, xs_ref, w_ref, b_ref, y_ref):
    y_ref[...] = (jnp.dot(xs_ref[...], w_ref[0],
                          preferred_element_type=jnp.float32) + b_ref[0])


def _stage_c(x_sorted, W_experts, b_experts, block_expert):
    grid_spec = pltpu.PrefetchScalarGridSpec(
        num_scalar_prefetch=1,
        grid=(NB,),
        in_specs=[
            pl.BlockSpec((BMG, HIDDEN), lambda i, be: (i, 0)),
            pl.BlockSpec((1, HIDDEN, HIDDEN), lambda i, be: (be[i], 0, 0)),
            pl.BlockSpec((1, 1, HIDDEN), lambda i, be: (be[i], 0, 0)),
        ],
        out_specs=pl.BlockSpec((BMG, HIDDEN), lambda i, be: (i, 0)),
    )
    return pl.pallas_call(
        _gmm_body,
        grid_spec=grid_spec,
        out_shape=jax.ShapeDtypeStruct((P_MAX, HIDDEN), jnp.float32),
        compiler_params=pltpu.CompilerParams(
            dimension_semantics=("arbitrary",),
        ),
    )(block_expert, x_sorted, W_experts,
      b_experts.reshape(NUM_EXPERTS, 1, HIDDEN))


TPW = N_TOKENS // (NC * NS)
VPR = HIDDEN // L


def _stage_d_kernel(gmt_hbm, dest_hbm, y_hbm, out_hbm,
                    gbuf, d0buf, d1buf, w1buf, w2buf,
                    y0buf, y1buf, obuf, y0buf2, y1buf2, obuf2,
                    semg0, semg1, semg0b, semg1b, semo0, semo1):
    c = lax.axis_index("c")
    s = lax.axis_index("s")
    wid = s * NC + c
    t0 = wid * TPW
    lanes = _lane_iota()

    for e in range(NUM_EXPERTS):
        pltpu.sync_copy(gmt_hbm.at[e, pl.ds(t0, TPW)], gbuf.at[e])
    pltpu.sync_copy(dest_hbm.at[pl.ds(t0, TPW)], d0buf)
    pltpu.sync_copy(dest_hbm.at[pl.ds(N_TOKENS + t0, TPW)], d1buf)

    def wts_body(i, carry):
        _, m1, m2 = _slot_expert(gbuf, i, 0)
        w1buf[pl.ds(i * L, L)] = m1
        w2buf[pl.ds(i * L, L)] = m2
        return carry

    lax.fori_loop(0, TPW // L, wts_body, 0, unroll=False)

    nch = TPW // L
    y0b = (y0buf, y0buf2)
    y1b = (y1buf, y1buf2)
    ob = (obuf, obuf2)
    sg0 = (semg0, semg0b)
    sg1 = (semg1, semg1b)
    so = (semo0, semo1)
    hg0 = [None, None]
    hg1 = [None, None]
    hout = [None, None]

    def start_gather(k, b):
        d016 = d0buf[pl.ds(k * L, L)]
        d116 = d1buf[pl.ds(k * L, L)]
        hg0[b] = pltpu.async_copy(y_hbm.at[d016], y0b[b], sg0[b])
        hg1[b] = pltpu.async_copy(y_hbm.at[d116], y1b[b], sg1[b])

    start_gather(0, 0)
    for k in range(nch):
        b = k % 2
        nb = (k + 1) % 2
        if k + 1 < nch:
            if hout[nb] is not None:
                hout[nb].wait()
            start_gather(k + 1, nb)
        hg0[b].wait()
        hg1[b].wait()
        w1v = w1buf[pl.ds(k * L, L)]
        w2v = w2buf[pl.ds(k * L, L)]
        for tl in range(L):
            w1 = _bc(w1v[tl], jnp.float32)
            w2 = _bc(w2v[tl], jnp.float32)

            def row_body(v, carry2, tl=tl, w1=w1, w2=w2, b=b):
                ob[b][tl, pl.ds(v * L, L)] = (
                    w1 * y0b[b][tl, pl.ds(v * L, L)]
                    + w2 * y1b[b][tl, pl.ds(v * L, L)])
                return carry2

            lax.fori_loop(0, VPR, row_body, 0, unroll=False)
        hout[b] = pltpu.async_copy(ob[b], out_hbm.at[pl.ds(t0 + k * L, L)],
                                   so[b])
    for h in hout:
        if h is not None:
            h.wait()


def _stage_d(g_mT, dest_pos, y):
    mesh = plsc.VectorSubcoreMesh(core_axis_name="c", subcore_axis_name="s")
    return pl.kernel(
        _stage_d_kernel,
        mesh=mesh,
        compiler_params=pltpu.CompilerParams(needs_layout_passes=False),
        out_type=[jax.ShapeDtypeStruct((N_TOKENS, HIDDEN), jnp.float32)],
        scratch_types=[
            pltpu.VMEM((NUM_EXPERTS, TPW), jnp.float32),
            pltpu.VMEM((TPW,), jnp.int32),
            pltpu.VMEM((TPW,), jnp.int32),
            pltpu.VMEM((TPW,), jnp.float32),
            pltpu.VMEM((TPW,), jnp.float32),
            pltpu.VMEM((L, HIDDEN), jnp.float32),
            pltpu.VMEM((L, HIDDEN), jnp.float32),
            pltpu.VMEM((L, HIDDEN), jnp.float32),
            pltpu.VMEM((L, HIDDEN), jnp.float32),
            pltpu.VMEM((L, HIDDEN), jnp.float32),
            pltpu.VMEM((L, HIDDEN), jnp.float32),
            pltpu.SemaphoreType.DMA,
            pltpu.SemaphoreType.DMA,
            pltpu.SemaphoreType.DMA,
            pltpu.SemaphoreType.DMA,
            pltpu.SemaphoreType.DMA,
            pltpu.SemaphoreType.DMA,
        ],
    )(g_mT, dest_pos, y)


_DEBUG_STAGE = ""
_NO_PHASE3 = False
_BCUT = 99


_JNP_SHUFFLE = False
_JNP_COMBINE = False


@jax.jit
def kernel(x, W_experts, b_experts, W_gate, b_gate):
    g_mT = _stage_a(x, W_gate, b_gate)
    dest_pos, block_expert, x_sorted, _hist_tab = _stage_b(g_mT, x)
    if _JNP_SHUFFLE:
        tok = jnp.concatenate([jnp.arange(N_TOKENS, dtype=jnp.int32)] * 2)
        x_sorted = jnp.zeros((P_MAX, HIDDEN), jnp.float32).at[dest_pos].set(
            x[tok])
    y = _stage_c(x_sorted, W_experts, b_experts, block_expert)
    if _JNP_COMBINE:
        gm = g_mT.T
        ids = jnp.arange(NUM_EXPERTS, dtype=jnp.int32)[None, :]
        m1 = jnp.max(gm, axis=-1)
        a1 = jnp.argmax(gm, axis=-1).astype(jnp.int32)
        gm2 = jnp.where(ids == a1[:, None], -jnp.inf, gm)
        m2 = jnp.max(gm2, axis=-1)
        return (m1[:, None] * y[dest_pos[:N_TOKENS]]
                + m2[:, None] * y[dest_pos[N_TOKENS:]])
    out = _stage_d(g_mT, dest_pos, y)
    return out[0] if isinstance(out, (list, tuple)) else out


# speedup vs baseline: 1.8085x; 1.0020x over previous
import jax
import jax.numpy as jnp
from jax import lax
from jax.experimental import pallas as pl
from jax.experimental.pallas import tpu as pltpu
from jax.experimental.pallas import tpu_sc as plsc

HIDDEN = 1024
NUM_EXPERTS = 8
TOP_K = 2
N_TOKENS = 4096

BMG = 256
P_MAX = 2 * N_TOKENS + NUM_EXPERTS * BMG
NB = P_MAX // BMG
NB_PAD = 64

NC = 2
NS = 16
L = 16
CH = (2 * N_TOKENS) // NS
NG = CH // L


def _topk_masked_gates(logits):
    m = jnp.max(logits, axis=-1, keepdims=True)
    ex = jnp.exp(logits - m)
    g = ex / jnp.sum(ex, axis=-1, keepdims=True)
    ids = jax.lax.broadcasted_iota(jnp.int32, g.shape, 1)
    m1 = jnp.max(g, axis=-1, keepdims=True)
    a1 = jnp.min(jnp.where(g == m1, ids, NUM_EXPERTS), axis=-1, keepdims=True)
    g_wo1 = jnp.where(ids == a1, -jnp.inf, g)
    m2 = jnp.max(g_wo1, axis=-1, keepdims=True)
    a2 = jnp.min(jnp.where(g_wo1 == m2, ids, NUM_EXPERTS), axis=-1,
                 keepdims=True)
    keep = (ids == a1) | (ids == a2)
    return jnp.where(keep, g, 0.0)


def _gate_body(x_ref, wg_ref, bg_ref, gmt_ref):
    logits = jnp.dot(x_ref[...], wg_ref[...],
                     preferred_element_type=jnp.float32) + bg_ref[...]
    gm = _topk_masked_gates(logits)
    gmt_ref[...] = gm.T


def _stage_a(x, W_gate, b_gate):
    bm = 1024
    return pl.pallas_call(
        _gate_body,
        grid=(N_TOKENS // bm,),
        in_specs=[
            pl.BlockSpec((bm, HIDDEN), lambda t: (t, 0)),
            pl.BlockSpec((HIDDEN, NUM_EXPERTS), lambda t: (0, 0)),
            pl.BlockSpec((1, NUM_EXPERTS), lambda t: (0, 0)),
        ],
        out_specs=pl.BlockSpec((NUM_EXPERTS, bm), lambda t: (0, t)),
        out_shape=jax.ShapeDtypeStruct((NUM_EXPERTS, N_TOKENS), jnp.float32),
    )(x, W_gate, b_gate.reshape(1, NUM_EXPERTS))


def _lane_iota():
    return jax.lax.broadcasted_iota(jnp.int32, (L,), 0)


def _bc(x, dtype=jnp.int32):
    return jnp.broadcast_to(jnp.asarray(x, dtype), (L,))


def _slot_expert(gbuf, i, slot):
    gs = [gbuf[e, pl.ds(i * L, L)] for e in range(NUM_EXPERTS)]
    m1 = gs[0]
    a1 = jnp.zeros((L,), jnp.int32)
    for e in range(1, NUM_EXPERTS):
        gt = gs[e] > m1
        m1 = jnp.where(gt, gs[e], m1)
        a1 = jnp.where(gt, _bc(e), a1)
    neg = jnp.full((L,), -jnp.inf, jnp.float32)
    m2 = jnp.where(a1 == jnp.zeros((L,), jnp.int32), neg, gs[0])
    a2 = jnp.zeros((L,), jnp.int32)
    for e in range(1, NUM_EXPERTS):
        ge = jnp.where(a1 == _bc(e), neg, gs[e])
        gt = ge > m2
        m2 = jnp.where(gt, ge, m2)
        a2 = jnp.where(gt, _bc(e), a2)
    ew = jnp.where(_bc(slot) == jnp.zeros((L,), jnp.int32), a1, a2)
    return ew, m1, m2


def _stage_b_kernel(gmt_hbm, x_hbm, dest_hbm, be_hbm, xs_hbm, hist_hbm,
                    gbuf, ebuf, dbuf, vbuf, tabbuf, bebuf,
                    rbuf0, rbuf1,
                    semg0, semg1, sems0, sems1):
    c = lax.axis_index("c")
    s = lax.axis_index("s")
    slot = s // 8
    t0 = (s % 8) * CH
    lanes = _lane_iota()

    for e in range(NUM_EXPERTS):
        pltpu.sync_copy(gmt_hbm.at[e, pl.ds(t0, CH)], gbuf.at[e])

    def p1_body(i, hist):
        ew, _, _ = _slot_expert(gbuf, i, slot)
        ebuf[pl.ds(i * L, L)] = ew
        for e in range(NUM_EXPERTS):
            ne = jnp.sum((ew == _bc(e)).astype(jnp.int32), axis=0)
            hist = hist + jnp.where(lanes == _bc(e), _bc(ne),
                                    jnp.zeros((L,), jnp.int32))
        return hist

    hist = lax.fori_loop(0, NG, p1_body, jnp.zeros((L,), jnp.int32),
                         unroll=False)
    vbuf[...] = hist
    pltpu.sync_copy(vbuf, hist_hbm.at[s])
    plsc.subcore_barrier()
    pltpu.sync_copy(hist_hbm, tabbuf)

    rows = [tabbuf[w] for w in range(NS)]
    tot = rows[0]
    for w in range(1, NS):
        tot = tot + rows[w]
    pad = jnp.bitwise_and(tot + (BMG - 1), -BMG)
    off = plsc.cumsum(pad) - pad
    base = off
    for w in range(NS):
        mk = _bc((w < s).astype(jnp.int32))
        base = base + rows[w] * mk

    zero_v = jnp.zeros((L,), jnp.int32)
    off_sc = [jnp.sum(jnp.where(lanes == _bc(e), off, zero_v), axis=0)
              for e in range(NUM_EXPERTS)]
    for k in range(NB_PAD // L):
        iv = (lanes + k * L) * BMG
        bev = jnp.full((L,), -1, jnp.int32)
        for e in range(NUM_EXPERTS):
            bev = bev + (iv >= _bc(off_sc[e])).astype(jnp.int32)
        bebuf[pl.ds(k * L, L)] = bev

    @pl.when(jnp.logical_and(c == 0, s == 0))
    def _():
        pltpu.sync_copy(bebuf, be_hbm)


    def p2_body(i, run):
        ew = ebuf[pl.ds(i * L, L)]
        dest = jnp.zeros((L,), jnp.int32)
        zv = jnp.zeros((L,), jnp.int32)
        for e in range(NUM_EXPERTS):
            mk = ew == _bc(e)
            mi = mk.astype(jnp.int32)
            pref = plsc.cumsum(mi) - mi
            run_e = jnp.sum(jnp.where(lanes == _bc(e), run, zv), axis=0)
            dest = jnp.where(mk, _bc(run_e) + pref, dest)
            run = run + jnp.where(lanes == _bc(e), _bc(jnp.sum(mi, axis=0)),
                                  zv)
        dbuf[pl.ds(i * L, L)] = dest
        return run

    lax.fori_loop(0, NG, p2_body, base, unroll=False)

    @pl.when(c == 0)
    def _():
        pltpu.sync_copy(dbuf, dest_hbm.at[pl.ds(s * CH, CH)])

    half = CH // 2
    tbase = t0 + c * half
    dofs = c * half
    rbufs = (rbuf0, rbuf1)
    semgs = (semg0, semg1)
    semss = (sems0, sems1)
    nch = half // L
    hg = [None, None]
    hscat = [None, None]
    hg[0] = pltpu.async_copy(x_hbm.at[pl.ds(tbase, L)], rbufs[0], semgs[0])
    for i in range(nch):
        b = i % 2
        nb = (i + 1) % 2
        if i + 1 < nch:
            if hscat[nb] is not None:
                hscat[nb].wait()
            hg[nb] = pltpu.async_copy(
                x_hbm.at[pl.ds(tbase + (i + 1) * L, L)], rbufs[nb],
                semgs[nb])
        hg[b].wait()
        dest16 = dbuf[pl.ds(dofs + i * L, L)]
        hscat[b] = pltpu.async_copy(rbufs[b], xs_hbm.at[dest16], semss[b])
    for h in hscat:
        if h is not None:
            h.wait()


def _stage_b(g_mT, x):
    mesh = plsc.VectorSubcoreMesh(core_axis_name="c", subcore_axis_name="s")
    return pl.kernel(
        _stage_b_kernel,
        mesh=mesh,
        compiler_params=pltpu.CompilerParams(needs_layout_passes=False),
        out_type=[
            jax.ShapeDtypeStruct((2 * N_TOKENS,), jnp.int32),
            jax.ShapeDtypeStruct((NB_PAD,), jnp.int32),
            jax.ShapeDtypeStruct((P_MAX, HIDDEN), jnp.float32),
            jax.ShapeDtypeStruct((NS, L), jnp.int32),
        ],
        scratch_types=[
            pltpu.VMEM((NUM_EXPERTS, CH), jnp.float32),
            pltpu.VMEM((CH,), jnp.int32),
            pltpu.VMEM((CH,), jnp.int32),
            pltpu.VMEM((L,), jnp.int32),
            pltpu.VMEM((NS, L), jnp.int32),
            pltpu.VMEM((NB_PAD,), jnp.int32),
            pltpu.VMEM((L, HIDDEN), jnp.float32),
            pltpu.VMEM((L, HIDDEN), jnp.float32),
            pltpu.SemaphoreType.DMA,
            pltpu.SemaphoreType.DMA,
            pltpu.SemaphoreType.DMA,
            pltpu.SemaphoreType.DMA,
        ],
    )(g_mT, x)


def _gmm_body(be_ref, xs_ref, w_ref, b_ref, y_ref):
    y_ref[...] = (jnp.dot(xs_ref[...], w_ref[0],
                          preferred_element_type=jnp.float32) + b_ref[0])


def _stage_c(x_sorted, W_experts, b_experts, block_expert):
    grid_spec = pltpu.PrefetchScalarGridSpec(
        num_scalar_prefetch=1,
        grid=(NB,),
        in_specs=[
            pl.BlockSpec((BMG, HIDDEN), lambda i, be: (i, 0)),
            pl.BlockSpec((1, HIDDEN, HIDDEN), lambda i, be: (be[i], 0, 0)),
            pl.BlockSpec((1, 1, HIDDEN), lambda i, be: (be[i], 0, 0)),
        ],
        out_specs=pl.BlockSpec((BMG, HIDDEN), lambda i, be: (i, 0)),
    )
    return pl.pallas_call(
        _gmm_body,
        grid_spec=grid_spec,
        out_shape=jax.ShapeDtypeStruct((P_MAX, HIDDEN), jnp.float32),
        compiler_params=pltpu.CompilerParams(
            dimension_semantics=("arbitrary",),
        ),
    )(block_expert, x_sorted, W_experts,
      b_experts.reshape(NUM_EXPERTS, 1, HIDDEN))


TPW = N_TOKENS // (NC * NS)
VPR = HIDDEN // L


def _stage_d_kernel(gmt_hbm, dest_hbm, y_hbm, out_hbm,
                    gbuf, d0buf, d1buf, w1buf, w2buf,
                    y0buf, y1buf, obuf, y0buf2, y1buf2, obuf2,
                    semg0, semg1, semg0b, semg1b, semo0, semo1):
    c = lax.axis_index("c")
    s = lax.axis_index("s")
    wid = s * NC + c
    t0 = wid * TPW
    lanes = _lane_iota()

    for e in range(NUM_EXPERTS):
        pltpu.sync_copy(gmt_hbm.at[e, pl.ds(t0, TPW)], gbuf.at[e])
    pltpu.sync_copy(dest_hbm.at[pl.ds(t0, TPW)], d0buf)
    pltpu.sync_copy(dest_hbm.at[pl.ds(N_TOKENS + t0, TPW)], d1buf)

    def wts_body(i, carry):
        _, m1, m2 = _slot_expert(gbuf, i, 0)
        w1buf[pl.ds(i * L, L)] = m1
        w2buf[pl.ds(i * L, L)] = m2
        return carry

    lax.fori_loop(0, TPW // L, wts_body, 0, unroll=False)

    nch = TPW // L
    y0b = (y0buf, y0buf2)
    y1b = (y1buf, y1buf2)
    ob = (obuf, obuf2)
    sg0 = (semg0, semg0b)
    sg1 = (semg1, semg1b)
    so = (semo0, semo1)
    hg0 = [None, None]
    hg1 = [None, None]
    hout = [None, None]

    def start_gather(k, b):
        d016 = d0buf[pl.ds(k * L, L)]
        d116 = d1buf[pl.ds(k * L, L)]
        hg0[b] = pltpu.async_copy(y_hbm.at[d016], y0b[b], sg0[b])
        hg1[b] = pltpu.async_copy(y_hbm.at[d116], y1b[b], sg1[b])

    start_gather(0, 0)
    for k in range(nch):
        b = k % 2
        nb = (k + 1) % 2
        if k + 1 < nch:
            if hout[nb] is not None:
                hout[nb].wait()
            start_gather(k + 1, nb)
        hg0[b].wait()
        hg1[b].wait()
        w1v = w1buf[pl.ds(k * L, L)]
        w2v = w2buf[pl.ds(k * L, L)]
        for tl in range(L):
            w1 = _bc(w1v[tl], jnp.float32)
            w2 = _bc(w2v[tl], jnp.float32)

            def row_body(v, carry2, tl=tl, w1=w1, w2=w2, b=b):
                ob[b][tl, pl.ds(v * L, L)] = (
                    w1 * y0b[b][tl, pl.ds(v * L, L)]
                    + w2 * y1b[b][tl, pl.ds(v * L, L)])
                return carry2

            lax.fori_loop(0, VPR, row_body, 0, unroll=False)
        hout[b] = pltpu.async_copy(ob[b], out_hbm.at[pl.ds(t0 + k * L, L)],
                                   so[b])
    for h in hout:
        if h is not None:
            h.wait()


def _stage_d(g_mT, dest_pos, y):
    mesh = plsc.VectorSubcoreMesh(core_axis_name="c", subcore_axis_name="s")
    return pl.kernel(
        _stage_d_kernel,
        mesh=mesh,
        compiler_params=pltpu.CompilerParams(needs_layout_passes=False),
        out_type=[jax.ShapeDtypeStruct((N_TOKENS, HIDDEN), jnp.float32)],
        scratch_types=[
            pltpu.VMEM((NUM_EXPERTS, TPW), jnp.float32),
            pltpu.VMEM((TPW,), jnp.int32),
            pltpu.VMEM((TPW,), jnp.int32),
            pltpu.VMEM((TPW,), jnp.float32),
            pltpu.VMEM((TPW,), jnp.float32),
            pltpu.VMEM((L, HIDDEN), jnp.float32),
            pltpu.VMEM((L, HIDDEN), jnp.float32),
            pltpu.VMEM((L, HIDDEN), jnp.float32),
            pltpu.VMEM((L, HIDDEN), jnp.float32),
            pltpu.VMEM((L, HIDDEN), jnp.float32),
            pltpu.VMEM((L, HIDDEN), jnp.float32),
            pltpu.SemaphoreType.DMA,
            pltpu.SemaphoreType.DMA,
            pltpu.SemaphoreType.DMA,
            pltpu.SemaphoreType.DMA,
            pltpu.SemaphoreType.DMA,
            pltpu.SemaphoreType.DMA,
        ],
    )(g_mT, dest_pos, y)


@jax.jit
def kernel(x, W_experts, b_experts, W_gate, b_gate):
    g_mT = _stage_a(x, W_gate, b_gate)
    dest_pos, block_expert, x_sorted, _hist_tab = _stage_b(g_mT, x)
    y = _stage_c(x_sorted, W_experts, b_experts, block_expert)
    out = _stage_d(g_mT, dest_pos, y)
    return out[0] if isinstance(out, (list, tuple)) else out
